# baseline (speedup 1.0000x reference)
import functools

import jax
import jax.numpy as jnp
from jax import lax
from jax.experimental import pallas as pl
from jax.experimental.pallas import tpu as pltpu
from jax.experimental.pallas import tpu_sc as plsc

T = 100
TP = 104
D = 768
B = 1024
NP = B * TP
NC = 2
NS = 16
NW = NC * NS
PROWS_W = NP // NW
BATCH_W = B // NW

CH_OFF = (0, 40, 80, 96)
CH_LEN = (40, 40, 16, 4)


def _combine_body(table_ref, mask_ref, out_ref):
    out_ref[0] = table_ref[...] * mask_ref[0]


def _gather_body(idx_hbm, comb_hbm, out_hbm, idx_v, cidx_v, *bufs_and_sems):
    bufs = bufs_and_sems[:4]
    sgs = bufs_and_sems[4:8]
    sss = bufs_and_sems[8:12]

    wid = lax.axis_index("s") * NC + lax.axis_index("c")
    pbase = wid * PROWS_W
    b0 = wid * BATCH_W

    pltpu.sync_copy(idx_hbm.at[pl.ds(pbase, PROWS_W)], idx_v)

    iota = lax.iota(jnp.int32, 16)

    def fold(k, carry):
        p = k * 16 + iota
        sl = pl.ds(k * 16, 16)
        cidx_v[sl] = idx_v[sl] + lax.rem(p, TP) * TP
        return carry

    lax.fori_loop(0, PROWS_W // 16, fold, 0)

    def gather(b_l, k):
        return pltpu.make_async_copy(
            comb_hbm.at[cidx_v.at[pl.ds(b_l * TP + CH_OFF[k], CH_LEN[k])]],
            bufs[k], sgs[k],
        )

    def scatter(b_l, k):
        return pltpu.make_async_copy(
            bufs[k], out_hbm.at[b0 + b_l, pl.ds(CH_OFF[k], CH_LEN[k])], sss[k]
        )

    for k in range(4):
        gather(0, k).start()

    def ring_body(b_l, carry):
        for k in range(4):
            gather(b_l, k).wait()
            scatter(b_l, k).start()

            @pl.when(b_l + 1 < BATCH_W)
            def _():
                scatter(b_l, k).wait()
                gather(b_l + 1, k).start()

        return carry

    lax.fori_loop(0, BATCH_W, ring_body, 0)
    for k in range(4):
        scatter(BATCH_W - 1, k).wait()


@jax.jit
def kernel(indices, embedding_weight, piece_mask):
    table_pad = jnp.pad(embedding_weight, ((0, TP - T), (0, 0)))
    mask3 = piece_mask[:, None, :]
    comb3 = pl.pallas_call(
        _combine_body,
        grid=(T,),
        in_specs=[
            pl.BlockSpec((TP, D), lambda t: (0, 0)),
            pl.BlockSpec((1, 1, D), lambda t: (t, 0, 0)),
        ],
        out_specs=pl.BlockSpec((1, TP, D), lambda t: (t, 0, 0)),
        out_shape=jax.ShapeDtypeStruct((T, TP, D), jnp.float32),
    )(table_pad, mask3)
    comb = comb3.reshape(T * TP, D)

    idx_p = jnp.concatenate(
        [indices, jnp.broadcast_to(indices[:, T - 1:T], (B, TP - T))], axis=1
    ).reshape(NP)

    mesh = plsc.VectorSubcoreMesh(core_axis_name="c", subcore_axis_name="s")
    run = functools.partial(
        pl.kernel,
        mesh=mesh,
        compiler_params=pltpu.CompilerParams(use_tc_tiling_on_sc=True),
        out_type=jax.ShapeDtypeStruct((B, T, D), jnp.float32),
        scratch_types=(
            [pltpu.VMEM((PROWS_W,), jnp.int32),
             pltpu.VMEM((PROWS_W,), jnp.int32)]
            + [pltpu.VMEM((n, D), jnp.float32) for n in CH_LEN]
            + [pltpu.SemaphoreType.DMA for _ in range(8)]
        ),
    )(_gather_body)
    return run(idx_p, comb)


# speedup vs baseline: 1.6694x; 1.6694x over previous
import functools

import jax
import jax.numpy as jnp
from jax import lax
from jax.experimental import pallas as pl
from jax.experimental.pallas import tpu as pltpu
from jax.experimental.pallas import tpu_sc as plsc

T = 100
TP = 104
D = 768
B = 1024
NP = B * TP
NC = 2
NS = 16
NW = NC * NS
FROWS_W = B * T // NW
CHUNK = 32
NCHUNK = FROWS_W // CHUNK
NBUF = 4


def _combine_body(table_ref, mask_ref, out_ref):
    out_ref[0] = table_ref[...] * mask_ref[0]


def _gather_body(idx_hbm, comb_hbm, out_hbm, idx_v, cidx_v, *bufs_and_sems):
    bufs = bufs_and_sems[:NBUF]
    sgs = bufs_and_sems[NBUF:2 * NBUF]
    sss = bufs_and_sems[2 * NBUF:3 * NBUF]

    wid = lax.axis_index("s") * NC + lax.axis_index("c")
    fbase = wid * FROWS_W

    pltpu.sync_copy(idx_hbm.at[pl.ds(fbase, FROWS_W)], idx_v)

    iota = lax.iota(jnp.int32, 16)

    def fold(k, carry):
        p = fbase + k * 16 + iota
        sl = pl.ds(k * 16, 16)
        cidx_v[sl] = idx_v[sl] + lax.shift_right_logical(p, 10) * TP
        return carry

    lax.fori_loop(0, FROWS_W // 16, fold, 0)

    def gather(c, k):
        return pltpu.make_async_copy(
            comb_hbm.at[cidx_v.at[pl.ds(c * CHUNK, CHUNK)]], bufs[k], sgs[k]
        )

    def scatter(c, k):
        r0 = fbase + c * CHUNK
        t = lax.shift_right_logical(r0, 10)
        b0 = lax.rem(r0, B)
        return pltpu.make_async_copy(
            bufs[k], out_hbm.at[t, pl.ds(b0, CHUNK)], sss[k]
        )

    for k in range(NBUF):
        gather(k, k).start()

    def ring_body(g, carry):
        for k in range(NBUF):
            c = g * NBUF + k
            gather(c, k).wait()
            scatter(c, k).start()

            @pl.when(c + NBUF < NCHUNK)
            def _():
                scatter(c, k).wait()
                gather(c + NBUF, k).start()

        return carry

    lax.fori_loop(0, NCHUNK // NBUF, ring_body, 0)
    for k in range(NBUF):
        scatter(NCHUNK - NBUF + k, k).wait()


@jax.jit
def kernel(indices, embedding_weight, piece_mask):
    table_pad = jnp.pad(embedding_weight, ((0, TP - T), (0, 0)))
    mask3 = piece_mask[:, None, :]
    comb3 = pl.pallas_call(
        _combine_body,
        grid=(T,),
        in_specs=[
            pl.BlockSpec((TP, D), lambda t: (0, 0)),
            pl.BlockSpec((1, 1, D), lambda t: (t, 0, 0)),
        ],
        out_specs=pl.BlockSpec((1, TP, D), lambda t: (t, 0, 0)),
        out_shape=jax.ShapeDtypeStruct((T, TP, D), jnp.float32),
    )(table_pad, mask3)
    comb = comb3.reshape(T * TP, D)

    idx_tf = indices.T.reshape(B * T)

    mesh = plsc.VectorSubcoreMesh(core_axis_name="c", subcore_axis_name="s")
    run = functools.partial(
        pl.kernel,
        mesh=mesh,
        out_type=jax.ShapeDtypeStruct((T, B, D), jnp.float32),
        scratch_types=(
            [pltpu.VMEM((FROWS_W,), jnp.int32),
             pltpu.VMEM((FROWS_W,), jnp.int32)]
            + [pltpu.VMEM((CHUNK, D), jnp.float32) for _ in range(NBUF)]
            + [pltpu.SemaphoreType.DMA for _ in range(2 * NBUF)]
        ),
    )(_gather_body)
    return run(idx_tf, comb).transpose(1, 0, 2)


# speedup vs baseline: 1.8714x; 1.1210x over previous
import functools

import jax
import jax.numpy as jnp
from jax import lax
from jax.experimental import pallas as pl
from jax.experimental.pallas import tpu as pltpu
from jax.experimental.pallas import tpu_sc as plsc

T = 100
TP = 104
D = 768
B = 1024
NP = B * TP
NC = 2
NS = 16
NW = NC * NS
FROWS_W = B * T // NW
CHUNK = 32
NCHUNK = FROWS_W // CHUNK
NBUF = 4


def _combine_body(table_ref, mask_ref, out_ref):
    out_ref[...] = table_ref[...] * mask_ref[:, 0, :][:, None, :]


def _gather_body(idx_hbm, comb_hbm, out_hbm, idx_v, cidx_v, *bufs_and_sems):
    bufs = bufs_and_sems[:NBUF]
    sgs = bufs_and_sems[NBUF:2 * NBUF]
    sss = bufs_and_sems[2 * NBUF:3 * NBUF]

    wid = lax.axis_index("s") * NC + lax.axis_index("c")
    fbase = wid * FROWS_W

    pltpu.sync_copy(idx_hbm.at[pl.ds(fbase, FROWS_W)], idx_v)

    iota = lax.iota(jnp.int32, 16)

    def fold(k, carry):
        p = fbase + k * 16 + iota
        sl = pl.ds(k * 16, 16)
        cidx_v[sl] = idx_v[sl] + lax.shift_right_logical(p, 10) * TP
        return carry

    lax.fori_loop(0, FROWS_W // 16, fold, 0)

    def gather(c, k):
        return pltpu.make_async_copy(
            comb_hbm.at[cidx_v.at[pl.ds(c * CHUNK, CHUNK)]], bufs[k], sgs[k]
        )

    def scatter(c, k):
        r0 = fbase + c * CHUNK
        t = lax.shift_right_logical(r0, 10)
        b0 = lax.rem(r0, B)
        return pltpu.make_async_copy(
            bufs[k], out_hbm.at[t, pl.ds(b0, CHUNK)], sss[k]
        )

    for k in range(NBUF):
        gather(k, k).start()

    def ring_body(g, carry):
        for k in range(NBUF):
            c = g * NBUF + k
            gather(c, k).wait()
            scatter(c, k).start()

            @pl.when(c + NBUF < NCHUNK)
            def _():
                scatter(c, k).wait()
                gather(c + NBUF, k).start()

        return carry

    lax.fori_loop(0, NCHUNK // NBUF, ring_body, 0)
    for k in range(NBUF):
        scatter(NCHUNK - NBUF + k, k).wait()


@jax.jit
def kernel(indices, embedding_weight, piece_mask):
    table_pad = jnp.pad(embedding_weight, ((0, TP - T), (0, 0)))
    mask3 = piece_mask[:, None, :]
    comb3 = pl.pallas_call(
        _combine_body,
        grid=(T // 4,),
        in_specs=[
            pl.BlockSpec((TP, D), lambda t: (0, 0)),
            pl.BlockSpec((4, 1, D), lambda t: (t, 0, 0)),
        ],
        out_specs=pl.BlockSpec((4, TP, D), lambda t: (t, 0, 0)),
        out_shape=jax.ShapeDtypeStruct((T, TP, D), jnp.float32),
    )(table_pad, mask3)
    comb = comb3.reshape(T * TP, D)

    idx_tf = indices.T.reshape(B * T)

    mesh = plsc.VectorSubcoreMesh(core_axis_name="c", subcore_axis_name="s")
    run = functools.partial(
        pl.kernel,
        mesh=mesh,
        out_type=jax.ShapeDtypeStruct((T, B, D), jnp.float32),
        scratch_types=(
            [pltpu.VMEM((FROWS_W,), jnp.int32),
             pltpu.VMEM((FROWS_W,), jnp.int32)]
            + [pltpu.VMEM((CHUNK, D), jnp.float32) for _ in range(NBUF)]
            + [pltpu.SemaphoreType.DMA for _ in range(2 * NBUF)]
        ),
    )(_gather_body)
    return run(idx_tf, comb).transpose(1, 0, 2)
